# baseline (speedup 1.0000x reference)
import functools

import jax
import jax.numpy as jnp
from jax import lax
from jax.experimental import pallas as pl
from jax.experimental.pallas import tpu as pltpu
from jax.experimental.pallas import tpu_sc as plsc

N = 10000
D = 128
E = 320000
EPS = 1e-5
NC, NS = 2, 16
NW = NC * NS
NPAD = 10240
RPT = NPAD // NS
K = 128
NB = 80
ET = NB * K
EPAD = NW * ET
TRASH = 10100
BLK = 1280
FBLK = 2000


def _mesh():
    return plsc.VectorSubcoreMesh(
        core_axis_name="c", subcore_axis_name="s", num_cores=NC, num_subcores=NS
    )


@functools.cache
def _build_deg():
    @functools.partial(
        pl.kernel,
        mesh=_mesh(),
        out_type=jax.ShapeDtypeStruct((NC, NPAD, D), jnp.float32),
        scratch_types=[
            pltpu.VMEM_SHARED((NPAD, D), jnp.float32),
            pltpu.VMEM((K, D), jnp.float32),
            pltpu.VMEM((K,), jnp.int32),
            pltpu.VMEM((K,), jnp.int32),
        ],
    )
    def deg_kernel(row_hbm, ones_hbm, zeros_hbm, degp_hbm, deg_sh, ones_v, pbuf, ridx):
        c = lax.axis_index("c")
        s = lax.axis_index("s")
        wid = c * NS + s
        base = s * RPT
        pltpu.sync_copy(zeros_hbm, deg_sh.at[pl.ds(base, RPT)])
        pltpu.sync_copy(ones_hbm, ones_v)
        plsc.subcore_barrier()

        def body(j, carry):
            pltpu.sync_copy(row_hbm.at[wid, j], ridx)
            pltpu.sync_copy(ones_v, deg_sh.at[ridx], add=True)
            return carry

        lax.fori_loop(0, NB, body, 0)
        plsc.subcore_barrier()
        pltpu.sync_copy(deg_sh.at[pl.ds(base, RPT)], degp_hbm.at[c, pl.ds(base, RPT)])

    return deg_kernel


@functools.cache
def _build_edge():
    @functools.partial(
        pl.kernel,
        mesh=_mesh(),
        out_type=jax.ShapeDtypeStruct((NC, NPAD, D), jnp.float32),
        scratch_types=[
            pltpu.VMEM_SHARED((NPAD, D), jnp.float32),
            pltpu.VMEM((K,), jnp.int32),
            pltpu.VMEM((K,), jnp.int32),
            pltpu.VMEM((K, D), jnp.float32),
            pltpu.SemaphoreType.DMA,
        ],
    )
    def edge_kernel(hw_hbm, row_hbm, col_hbm, zeros_hbm, acc_hbm, acc_sh,
                    ridx, cidx, gbuf, gsem):
        c = lax.axis_index("c")
        s = lax.axis_index("s")
        wid = c * NS + s
        base = s * RPT

        @pl.when(c == 0)
        def _():
            pltpu.sync_copy(hw_hbm.at[pl.ds(base, RPT)], acc_sh.at[pl.ds(base, RPT)])

        @pl.when(c == 1)
        def _():
            pltpu.sync_copy(zeros_hbm, acc_sh.at[pl.ds(base, RPT)])

        plsc.subcore_barrier()

        def body(j, carry):
            pltpu.sync_copy(row_hbm.at[wid, j], ridx)
            pltpu.sync_copy(col_hbm.at[wid, j], cidx)
            pltpu.async_copy(hw_hbm.at[ridx], gbuf, gsem).wait()
            pltpu.sync_copy(gbuf, acc_sh.at[cidx], add=True)
            return carry

        lax.fori_loop(0, NB, body, 0)
        plsc.subcore_barrier()
        pltpu.sync_copy(acc_sh.at[pl.ds(base, RPT)], acc_hbm.at[c, pl.ds(base, RPT)])

    return edge_kernel


def _dis_of(degp_ref):
    deg = degp_ref[0, :, 0:1] + degp_ref[1, :, 0:1] + 1.0
    return lax.rsqrt(deg)


def _h_from_acc(acc_ref, degp_ref, bprev_ref):
    h = acc_ref[0] + acc_ref[1]
    return h * _dis_of(degp_ref) + bprev_ref[...]


def _stats_accumulate(h, i, out_ref):
    rows = i * BLK + lax.broadcasted_iota(jnp.int32, (BLK, 1), 0)
    hm = jnp.where(rows < N, h, 0.0)

    @pl.when(i == 0)
    def _():
        out_ref[...] = jnp.zeros_like(out_ref)

    out_ref[0:1, :] += jnp.sum(hm, axis=0, keepdims=True)
    out_ref[1:2, :] += jnp.sum(hm * hm, axis=0, keepdims=True)


def _apply_body(h, degp_ref, stats_ref, g_ref, bb_ref, w_ref, out_ref):
    mean = stats_ref[0:1, :] * (1.0 / N)
    ex2 = stats_ref[1:2, :] * (1.0 / N)
    rstd = lax.rsqrt(ex2 - mean * mean + EPS)
    hn = (h - mean) * (rstd * g_ref[...]) + bb_ref[...]
    hwn = jnp.dot(hn, w_ref[...], preferred_element_type=jnp.float32,
                  precision=lax.Precision.HIGHEST)
    out_ref[...] = hwn * _dis_of(degp_ref)


def _build_stats0(interpret=False):
    def body(x_ref, out_ref):
        _stats_accumulate(x_ref[...], pl.program_id(0), out_ref)

    return pl.pallas_call(
        body,
        grid=(NPAD // BLK,),
        in_specs=[pl.BlockSpec((BLK, D), lambda i: (i, 0))],
        out_specs=pl.BlockSpec((8, D), lambda i: (0, 0)),
        out_shape=jax.ShapeDtypeStruct((8, D), jnp.float32),
        interpret=interpret,
    )


def _build_statsk(interpret=False):
    def body(acc_ref, degp_ref, bprev_ref, out_ref):
        h = _h_from_acc(acc_ref, degp_ref, bprev_ref)
        _stats_accumulate(h, pl.program_id(0), out_ref)

    return pl.pallas_call(
        body,
        grid=(NPAD // BLK,),
        in_specs=[
            pl.BlockSpec((NC, BLK, D), lambda i: (0, i, 0)),
            pl.BlockSpec((NC, BLK, D), lambda i: (0, i, 0)),
            pl.BlockSpec((1, D), lambda i: (0, 0)),
        ],
        out_specs=pl.BlockSpec((8, D), lambda i: (0, 0)),
        out_shape=jax.ShapeDtypeStruct((8, D), jnp.float32),
        interpret=interpret,
    )


def _build_apply0(interpret=False):
    def body(x_ref, degp_ref, stats_ref, g_ref, bb_ref, w_ref, out_ref):
        _apply_body(x_ref[...], degp_ref, stats_ref, g_ref, bb_ref, w_ref,
                    out_ref)

    return pl.pallas_call(
        body,
        grid=(NPAD // BLK,),
        in_specs=[
            pl.BlockSpec((BLK, D), lambda i: (i, 0)),
            pl.BlockSpec((NC, BLK, D), lambda i: (0, i, 0)),
            pl.BlockSpec((8, D), lambda i: (0, 0)),
            pl.BlockSpec((1, D), lambda i: (0, 0)),
            pl.BlockSpec((1, D), lambda i: (0, 0)),
            pl.BlockSpec((D, D), lambda i: (0, 0)),
        ],
        out_specs=pl.BlockSpec((BLK, D), lambda i: (i, 0)),
        out_shape=jax.ShapeDtypeStruct((NPAD, D), jnp.float32),
        interpret=interpret,
    )


def _build_applyk(interpret=False):
    def body(acc_ref, degp_ref, stats_ref, g_ref, bb_ref, bprev_ref, w_ref,
             out_ref):
        h = _h_from_acc(acc_ref, degp_ref, bprev_ref)
        _apply_body(h, degp_ref, stats_ref, g_ref, bb_ref, w_ref, out_ref)

    return pl.pallas_call(
        body,
        grid=(NPAD // BLK,),
        in_specs=[
            pl.BlockSpec((NC, BLK, D), lambda i: (0, i, 0)),
            pl.BlockSpec((NC, BLK, D), lambda i: (0, i, 0)),
            pl.BlockSpec((8, D), lambda i: (0, 0)),
            pl.BlockSpec((1, D), lambda i: (0, 0)),
            pl.BlockSpec((1, D), lambda i: (0, 0)),
            pl.BlockSpec((1, D), lambda i: (0, 0)),
            pl.BlockSpec((D, D), lambda i: (0, 0)),
        ],
        out_specs=pl.BlockSpec((BLK, D), lambda i: (i, 0)),
        out_shape=jax.ShapeDtypeStruct((NPAD, D), jnp.float32),
        interpret=interpret,
    )


def _build_final(interpret=False):
    def body(acc_ref, degp_ref, b_ref, out_ref):
        out_ref[...] = _h_from_acc(acc_ref, degp_ref, b_ref)

    return pl.pallas_call(
        body,
        grid=(N // FBLK,),
        in_specs=[
            pl.BlockSpec((NC, FBLK, D), lambda i: (0, i, 0)),
            pl.BlockSpec((NC, FBLK, D), lambda i: (0, i, 0)),
            pl.BlockSpec((1, D), lambda i: (0, 0)),
        ],
        out_specs=pl.BlockSpec((FBLK, D), lambda i: (i, 0)),
        out_shape=jax.ShapeDtypeStruct((N, D), jnp.float32),
        interpret=interpret,
    )


_STATS0 = _build_stats0()
_STATSK = _build_statsk()
_APPLY0 = _build_apply0()
_APPLYK = _build_applyk()
_FINAL = _build_final()


def kernel(x, edge_index, bn0_g, bn0_b, W0, b0, Ws, bs, gammas, betas):
    _DEG = _build_deg()
    _EDGE = _build_edge()
    pad = jnp.full((EPAD - E,), TRASH, jnp.int32)
    row_t = jnp.concatenate([edge_index[0], pad]).reshape(NW, NB, K)
    col_t = jnp.concatenate([edge_index[1], pad]).reshape(NW, NB, K)
    ones_tab = jnp.ones((K, D), jnp.float32)
    zeros_tab = jnp.zeros((RPT, D), jnp.float32)
    degp = _DEG(row_t, ones_tab, zeros_tab)
    x_pad = jnp.pad(x, ((0, NPAD - N), (0, 0)))

    stats = _STATS0(x_pad)
    hw = _APPLY0(x_pad, degp, stats, bn0_g.reshape(1, D), bn0_b.reshape(1, D),
                 W0)
    acc = _EDGE(hw, row_t, col_t, zeros_tab)
    prev_b = b0
    for i in range(3):
        pb = prev_b.reshape(1, D)
        stats = _STATSK(acc, degp, pb)
        hw = _APPLYK(acc, degp, stats, gammas[i].reshape(1, D),
                     betas[i].reshape(1, D), pb, Ws[i])
        acc = _EDGE(hw, row_t, col_t, zeros_tab)
        prev_b = bs[i]
    return _FINAL(acc, degp, prev_b.reshape(1, D))


# speedup vs baseline: 1.4023x; 1.4023x over previous
import functools

import jax
import jax.numpy as jnp
from jax import lax
from jax.experimental import pallas as pl
from jax.experimental.pallas import tpu as pltpu
from jax.experimental.pallas import tpu_sc as plsc

N = 10000
D = 128
E = 320000
EPS = 1e-5
NC, NS = 2, 16
NW = NC * NS
NPAD = 10240
RPT = NPAD // NS
K = 128
NB = 79
ET = NB * K
EPAD = NW * ET
TRASH = 10100
BLK = 1280
FBLK = 2000


def _mesh():
    return plsc.VectorSubcoreMesh(
        core_axis_name="c", subcore_axis_name="s", num_cores=NC, num_subcores=NS
    )


@functools.cache
def _build_deg():
    @functools.partial(
        pl.kernel,
        mesh=_mesh(),
        out_type=jax.ShapeDtypeStruct((NC, NPAD, D), jnp.float32),
        scratch_types=[
            pltpu.VMEM_SHARED((NPAD, D), jnp.float32),
            pltpu.VMEM((K, D), jnp.float32),
            pltpu.VMEM((K,), jnp.int32),
            pltpu.VMEM((K,), jnp.int32),
        ],
    )
    def deg_kernel(row_hbm, ones_hbm, zeros_hbm, degp_hbm, deg_sh, ones_v, pbuf, ridx):
        c = lax.axis_index("c")
        s = lax.axis_index("s")
        wid = c * NS + s
        base = s * RPT
        pltpu.sync_copy(zeros_hbm, deg_sh.at[pl.ds(base, RPT)])
        pltpu.sync_copy(ones_hbm, ones_v)
        plsc.subcore_barrier()

        def body(j, carry):
            pltpu.sync_copy(row_hbm.at[wid, j], ridx)
            pltpu.sync_copy(ones_v, deg_sh.at[ridx], add=True)
            return carry

        lax.fori_loop(0, NB, body, 0)
        plsc.subcore_barrier()
        pltpu.sync_copy(deg_sh.at[pl.ds(base, RPT)], degp_hbm.at[c, pl.ds(base, RPT)])

    return deg_kernel


@functools.cache
def _build_edge():
    @functools.partial(
        pl.kernel,
        mesh=_mesh(),
        out_type=jax.ShapeDtypeStruct((NC, NPAD, D), jnp.float32),
        scratch_types=[
            pltpu.VMEM_SHARED((NPAD, D), jnp.float32),
            pltpu.VMEM((K,), jnp.int32),
            pltpu.VMEM((K,), jnp.int32),
            pltpu.VMEM((K, D), jnp.float32),
            pltpu.SemaphoreType.DMA,
        ],
    )
    def edge_kernel(hw_hbm, row_hbm, col_hbm, zeros_hbm, acc_hbm, acc_sh,
                    ridx, cidx, gbuf, gsem):
        c = lax.axis_index("c")
        s = lax.axis_index("s")
        wid = c * NS + s
        base = s * RPT

        @pl.when(c == 0)
        def _():
            pltpu.sync_copy(hw_hbm.at[pl.ds(base, RPT)], acc_sh.at[pl.ds(base, RPT)])

        @pl.when(c == 1)
        def _():
            pltpu.sync_copy(zeros_hbm, acc_sh.at[pl.ds(base, RPT)])

        plsc.subcore_barrier()

        def body(j, carry):
            pltpu.sync_copy(row_hbm.at[wid, j], ridx)
            pltpu.sync_copy(col_hbm.at[wid, j], cidx)
            pltpu.async_copy(hw_hbm.at[ridx], gbuf, gsem).wait()
            pltpu.sync_copy(gbuf, acc_sh.at[cidx], add=True)
            return carry

        lax.fori_loop(0, NB, body, 0)
        plsc.subcore_barrier()
        pltpu.sync_copy(acc_sh.at[pl.ds(base, RPT)], acc_hbm.at[c, pl.ds(base, RPT)])

    return edge_kernel


def _dis_of(degp_ref):
    deg = degp_ref[0, :, 0:1] + degp_ref[1, :, 0:1] + 1.0
    return lax.rsqrt(deg)


def _h_from_acc(acc_ref, degp_ref, bprev_ref):
    h = acc_ref[0] + acc_ref[1]
    return h * _dis_of(degp_ref) + bprev_ref[...]


def _stats_accumulate(h, i, out_ref):
    rows = i * BLK + lax.broadcasted_iota(jnp.int32, (BLK, 1), 0)
    hm = jnp.where(rows < N, h, 0.0)

    @pl.when(i == 0)
    def _():
        out_ref[...] = jnp.zeros_like(out_ref)

    out_ref[0:1, :] += jnp.sum(hm, axis=0, keepdims=True)
    out_ref[1:2, :] += jnp.sum(hm * hm, axis=0, keepdims=True)


def _apply_body(h, degp_ref, stats_ref, g_ref, bb_ref, w_ref, out_ref):
    mean = stats_ref[0:1, :] * (1.0 / N)
    ex2 = stats_ref[1:2, :] * (1.0 / N)
    rstd = lax.rsqrt(ex2 - mean * mean + EPS)
    hn = (h - mean) * (rstd * g_ref[...]) + bb_ref[...]
    hwn = jnp.dot(hn, w_ref[...], preferred_element_type=jnp.float32,
                  precision=lax.Precision.HIGHEST)
    out_ref[...] = hwn * _dis_of(degp_ref)


def _build_stats0(interpret=False):
    def body(x_ref, out_ref):
        _stats_accumulate(x_ref[...], pl.program_id(0), out_ref)

    return pl.pallas_call(
        body,
        grid=(NPAD // BLK,),
        in_specs=[pl.BlockSpec((BLK, D), lambda i: (i, 0))],
        out_specs=pl.BlockSpec((8, D), lambda i: (0, 0)),
        out_shape=jax.ShapeDtypeStruct((8, D), jnp.float32),
        interpret=interpret,
    )


def _build_statsk(interpret=False):
    def body(acc_ref, degp_ref, bprev_ref, out_ref):
        h = _h_from_acc(acc_ref, degp_ref, bprev_ref)
        _stats_accumulate(h, pl.program_id(0), out_ref)

    return pl.pallas_call(
        body,
        grid=(NPAD // BLK,),
        in_specs=[
            pl.BlockSpec((NC, BLK, D), lambda i: (0, i, 0)),
            pl.BlockSpec((NC, BLK, D), lambda i: (0, i, 0)),
            pl.BlockSpec((1, D), lambda i: (0, 0)),
        ],
        out_specs=pl.BlockSpec((8, D), lambda i: (0, 0)),
        out_shape=jax.ShapeDtypeStruct((8, D), jnp.float32),
        interpret=interpret,
    )


def _build_apply0(interpret=False):
    def body(x_ref, degp_ref, stats_ref, g_ref, bb_ref, w_ref, out_ref):
        _apply_body(x_ref[...], degp_ref, stats_ref, g_ref, bb_ref, w_ref,
                    out_ref)

    return pl.pallas_call(
        body,
        grid=(NPAD // BLK,),
        in_specs=[
            pl.BlockSpec((BLK, D), lambda i: (i, 0)),
            pl.BlockSpec((NC, BLK, D), lambda i: (0, i, 0)),
            pl.BlockSpec((8, D), lambda i: (0, 0)),
            pl.BlockSpec((1, D), lambda i: (0, 0)),
            pl.BlockSpec((1, D), lambda i: (0, 0)),
            pl.BlockSpec((D, D), lambda i: (0, 0)),
        ],
        out_specs=pl.BlockSpec((BLK, D), lambda i: (i, 0)),
        out_shape=jax.ShapeDtypeStruct((NPAD, D), jnp.float32),
        interpret=interpret,
    )


def _build_applyk(interpret=False):
    def body(acc_ref, degp_ref, stats_ref, g_ref, bb_ref, bprev_ref, w_ref,
             out_ref):
        h = _h_from_acc(acc_ref, degp_ref, bprev_ref)
        _apply_body(h, degp_ref, stats_ref, g_ref, bb_ref, w_ref, out_ref)

    return pl.pallas_call(
        body,
        grid=(NPAD // BLK,),
        in_specs=[
            pl.BlockSpec((NC, BLK, D), lambda i: (0, i, 0)),
            pl.BlockSpec((NC, BLK, D), lambda i: (0, i, 0)),
            pl.BlockSpec((8, D), lambda i: (0, 0)),
            pl.BlockSpec((1, D), lambda i: (0, 0)),
            pl.BlockSpec((1, D), lambda i: (0, 0)),
            pl.BlockSpec((1, D), lambda i: (0, 0)),
            pl.BlockSpec((D, D), lambda i: (0, 0)),
        ],
        out_specs=pl.BlockSpec((BLK, D), lambda i: (i, 0)),
        out_shape=jax.ShapeDtypeStruct((NPAD, D), jnp.float32),
        interpret=interpret,
    )


def _build_final(interpret=False):
    def body(acc_ref, degp_ref, b_ref, out_ref):
        out_ref[...] = _h_from_acc(acc_ref, degp_ref, b_ref)

    return pl.pallas_call(
        body,
        grid=(N // FBLK,),
        in_specs=[
            pl.BlockSpec((NC, FBLK, D), lambda i: (0, i, 0)),
            pl.BlockSpec((NC, FBLK, D), lambda i: (0, i, 0)),
            pl.BlockSpec((1, D), lambda i: (0, 0)),
        ],
        out_specs=pl.BlockSpec((FBLK, D), lambda i: (i, 0)),
        out_shape=jax.ShapeDtypeStruct((N, D), jnp.float32),
        interpret=interpret,
    )


_STATS0 = _build_stats0()
_STATSK = _build_statsk()
_APPLY0 = _build_apply0()
_APPLYK = _build_applyk()
_FINAL = _build_final()


def kernel(x, edge_index, bn0_g, bn0_b, W0, b0, Ws, bs, gammas, betas):
    _DEG = _build_deg()
    _EDGE = _build_edge()
    pad = jnp.full((EPAD - E,), TRASH, jnp.int32)
    row_t = jnp.concatenate([edge_index[0], pad]).reshape(NW, NB, K)
    col_t = jnp.concatenate([edge_index[1], pad]).reshape(NW, NB, K)
    ones_tab = jnp.ones((K, D), jnp.float32)
    zeros_tab = jnp.zeros((RPT, D), jnp.float32)
    degp = _DEG(row_t, ones_tab, zeros_tab)
    x_pad = jnp.pad(x, ((0, NPAD - N), (0, 0)))

    stats = _STATS0(x_pad)
    hw = _APPLY0(x_pad, degp, stats, bn0_g.reshape(1, D), bn0_b.reshape(1, D),
                 W0)
    acc = _EDGE(hw, row_t, col_t, zeros_tab)
    prev_b = b0
    for i in range(3):
        pb = prev_b.reshape(1, D)
        stats = _STATSK(acc, degp, pb)
        hw = _APPLYK(acc, degp, stats, gammas[i].reshape(1, D),
                     betas[i].reshape(1, D), pb, Ws[i])
        acc = _EDGE(hw, row_t, col_t, zeros_tab)
        prev_b = bs[i]
    return _FINAL(acc, degp, prev_b.reshape(1, D))
